# baseline (speedup 1.0000x reference)
import functools

import jax
import jax.numpy as jnp
from jax import lax
from jax.experimental import pallas as pl
from jax.experimental.pallas import tpu as pltpu
from jax.experimental.pallas import tpu_sc as plsc

NC = 2
NS = 16
CH = 128


def _sc_mesh():
    return plsc.VectorSubcoreMesh(core_axis_name="c", subcore_axis_name="s")


@functools.lru_cache(maxsize=None)
def _make_deg_kernel(NP: int, n_chunks: int, D: int):
    rows = NP // NS

    @functools.partial(
        pl.kernel,
        out_type=jax.ShapeDtypeStruct((2, NP, D), jnp.float32),
        mesh=_sc_mesh(),
        scratch_types=[
            pltpu.VMEM((n_chunks, CH), jnp.int32),
            pltpu.VMEM((CH, D), jnp.float32),
            pltpu.VMEM_SHARED((NP, D), jnp.float32),
            pltpu.SemaphoreType.DMA,
        ],
    )
    def deg_kernel(edges_h, zeros_h, ones_h, out_h, idx_v, val_v, acc_sh, sem):
        c = lax.axis_index("c")
        s = lax.axis_index("s")
        pltpu.sync_copy(zeros_h, val_v)
        off = 0
        while off < rows:
            n = min(CH, rows - off)
            pltpu.sync_copy(val_v.at[pl.ds(0, n)],
                            acc_sh.at[pl.ds(s * rows + off, n)])
            off += n
        pltpu.sync_copy(edges_h.at[c, s], idx_v)
        pltpu.sync_copy(ones_h, val_v)
        plsc.subcore_barrier()

        K = 4

        for j in range(K):
            pltpu.async_copy(val_v, acc_sh.at[idx_v.at[j]], sem, add=True)

        def body(j, carry):
            pltpu.make_async_copy(
                val_v, acc_sh.at[idx_v.at[0]], sem).wait()
            pltpu.async_copy(val_v, acc_sh.at[idx_v.at[j + K]], sem, add=True)
            return carry

        lax.fori_loop(0, n_chunks - K, body, 0)
        for j in range(K):
            pltpu.make_async_copy(
                val_v, acc_sh.at[idx_v.at[0]], sem).wait()
        plsc.subcore_barrier()
        pltpu.sync_copy(acc_sh.at[pl.ds(s * rows, rows)],
                        out_h.at[c, pl.ds(s * rows, rows)])

    return deg_kernel


@functools.lru_cache(maxsize=None)
def _make_agg_kernel(NP: int, n_chunks: int, D: int):
    rows = NP // NS
    IB = min(n_chunks, 40)
    blocks = []
    st = 0
    while st < n_chunks:
        blocks.append((st, min(IB, n_chunks - st)))
        st += IB

    @functools.partial(
        pl.kernel,
        out_type=jax.ShapeDtypeStruct((2, NP, D), jnp.float32),
        mesh=_sc_mesh(),
        scratch_types=[
            pltpu.VMEM((IB, CH), jnp.int32),
            pltpu.VMEM((IB, CH), jnp.int32),
            pltpu.VMEM((CH, D), jnp.float32),
            pltpu.VMEM((CH, D), jnp.float32),
            pltpu.VMEM_SHARED((NP, D), jnp.float32),
            pltpu.SemaphoreType.DMA,
            pltpu.SemaphoreType.DMA,
            pltpu.SemaphoreType.DMA,
            pltpu.SemaphoreType.DMA,
        ],
    )
    def agg_kernel(table_h, src_h, dst_h, zeros_h, out_h,
                   si_v, di_v, b0_v, b1_v, acc_sh, gs0, gs1, ss0, ss1):
        c = lax.axis_index("c")
        s = lax.axis_index("s")
        wid = c * NS + s
        pltpu.sync_copy(zeros_h, b0_v)
        off = 0
        while off < rows:
            n = min(CH, rows - off)
            pltpu.sync_copy(b0_v.at[pl.ds(0, n)],
                            acc_sh.at[pl.ds(s * rows + off, n)])
            off += n
        plsc.subcore_barrier()

        bufs = (b0_v, b1_v)
        gsems = (gs0, gs1)
        ssems = (ss0, ss1)

        def wait_g(b):
            pltpu.make_async_copy(
                table_h.at[si_v.at[b]], bufs[b], gsems[b]).wait()

        def wait_s(b):
            pltpu.make_async_copy(
                bufs[b], acc_sh.at[di_v.at[b]], ssems[b]).wait()

        for st_, nb in blocks:
            pltpu.sync_copy(src_h.at[wid, pl.ds(st_, nb)], si_v.at[pl.ds(0, nb)])
            pltpu.sync_copy(dst_h.at[wid, pl.ds(st_, nb)], di_v.at[pl.ds(0, nb)])
            for b in range(2):
                pltpu.async_copy(table_h.at[si_v.at[b]], bufs[b], gsems[b])

            def body(j, carry):
                ch0 = j * 2
                for b in range(2):
                    wait_g(b)
                    pltpu.async_copy(
                        bufs[b], acc_sh.at[di_v.at[ch0 + b]], ssems[b],
                        add=True)
                for b in range(2):
                    wait_s(b)
                    pltpu.async_copy(
                        table_h.at[si_v.at[ch0 + b + 2]], bufs[b], gsems[b])
                return carry

            lax.fori_loop(0, nb // 2 - 1, body, 0)
            for b in range(2):
                wait_g(b)
                pltpu.async_copy(
                    bufs[b], acc_sh.at[di_v.at[nb - 2 + b]], ssems[b],
                    add=True)
            for b in range(2):
                wait_s(b)
        plsc.subcore_barrier()
        pltpu.sync_copy(acc_sh.at[pl.ds(s * rows, rows)],
                        out_h.at[c, pl.ds(s * rows, rows)])

    return agg_kernel


def _norm_col(deg_blk):
    d = deg_blk[:, 0:1]
    return jnp.where(d > 0, lax.rsqrt(jnp.maximum(d, 1e-12)), 0.0)


def _leaky(h):
    return jnp.where(h >= 0, h, 0.01 * h)


def _xs_body(x_ref, deg_ref, o_ref):
    o_ref[...] = x_ref[...] * _norm_col(deg_ref[...])


def _m1_body(agg_ref, degs_ref, w_ref, b_ref, o_ref):
    a = agg_ref[0] + agg_ref[1]
    degs = degs_ref[...]
    nd = jnp.where(degs[1][:, 0:1] > 0,
                   lax.rsqrt(jnp.maximum(degs[1][:, 0:1], 1e-12)), 0.0)
    ns = jnp.where(degs[0][:, 0:1] > 0,
                   lax.rsqrt(jnp.maximum(degs[0][:, 0:1], 1e-12)), 0.0)
    h = jnp.dot(a * nd, w_ref[...], preferred_element_type=jnp.float32)
    o_ref[...] = _leaky(h + b_ref[...]) * ns


def _m2_body(agg_ref, degs_ref, w_ref, b_ref, wl_ref, bl_ref, o_ref):
    a = agg_ref[0] + agg_ref[1]
    degs = degs_ref[...]
    nd = jnp.where(degs[1][:, 0:1] > 0,
                   lax.rsqrt(jnp.maximum(degs[1][:, 0:1], 1e-12)), 0.0)
    h = _leaky(jnp.dot(a * nd, w_ref[...],
                       preferred_element_type=jnp.float32) + b_ref[...])
    o_ref[...] = jnp.dot(h, wl_ref[...],
                         preferred_element_type=jnp.float32) + bl_ref[...]


@functools.lru_cache(maxsize=None)
def _make_tc_kernels(NP: int, D: int):
    R = NP // 16
    grid = (NP // R,)
    row_spec = pl.BlockSpec((R, D), lambda i: (i, 0))
    deg1_spec = pl.BlockSpec((R, D), lambda i: (i, 0))
    degs_spec = pl.BlockSpec((2, R, D), lambda i: (0, i, 0))
    agg_spec = pl.BlockSpec((2, R, D), lambda i: (0, i, 0))
    w_spec = pl.BlockSpec((D, D), lambda i: (0, 0))
    b_spec = pl.BlockSpec((1, D), lambda i: (0, 0))
    out = jax.ShapeDtypeStruct((NP, D), jnp.float32)

    xs_call = pl.pallas_call(
        _xs_body, grid=grid, in_specs=[row_spec, deg1_spec],
        out_specs=row_spec, out_shape=out)
    m1_call = pl.pallas_call(
        _m1_body, grid=grid,
        in_specs=[agg_spec, degs_spec, w_spec, b_spec],
        out_specs=row_spec, out_shape=out)
    m2_call = pl.pallas_call(
        _m2_body, grid=grid,
        in_specs=[agg_spec, degs_spec, w_spec, b_spec, w_spec, b_spec],
        out_specs=row_spec, out_shape=out)
    return xs_call, m1_call, m2_call


def kernel(x, edge_index, W1, b1, W2, b2, Wl, bl):
    N, D = x.shape
    E = edge_index.shape[1]
    C = Wl.shape[1]

    NP = ((N + 1 + 127) // 128) * 128
    EPAD = ((E + 2 * NC * NS * CH - 1) // (2 * NC * NS * CH)) * (2 * NC * NS * CH)
    n_agg = EPAD // (NC * NS * CH)
    n_deg = EPAD // (NS * CH)

    pad = jnp.full((2, EPAD - E), N, jnp.int32)
    e_pad = jnp.concatenate([edge_index, pad], axis=1)
    e_deg = e_pad.reshape(2, NS, n_deg, CH)
    src_agg = e_pad[0].reshape(NC * NS, n_agg, CH)
    dst_agg = e_pad[1].reshape(NC * NS, n_agg, CH)

    xp = jnp.zeros((NP, D), jnp.float32).at[:N].set(x)
    ones_c = jnp.ones((CH, D), jnp.float32)
    zeros_c = jnp.zeros((CH, D), jnp.float32)
    b1p = b1.reshape(1, D)
    b2p = b2.reshape(1, D)
    Wlp = jnp.zeros((D, D), jnp.float32).at[:, :C].set(Wl)
    blp = jnp.zeros((1, D), jnp.float32).at[0, :C].set(bl)

    deg_k = _make_deg_kernel(NP, n_deg, D)
    agg_k = _make_agg_kernel(NP, n_agg, D)
    xs_call, m1_call, m2_call = _make_tc_kernels(NP, D)

    degs = deg_k(e_deg, zeros_c, ones_c)
    xs = xs_call(xp, degs[0])
    agg1 = agg_k(xs, src_agg, dst_agg, zeros_c)
    h1s = m1_call(agg1, degs, W1, b1p)
    agg2 = agg_k(h1s, src_agg, dst_agg, zeros_c)
    outp = m2_call(agg2, degs, W2, b2p, Wlp, blp)
    return outp[:N, :C]


# speedup vs baseline: 2.4694x; 2.4694x over previous
import functools

import jax
import jax.numpy as jnp
from jax import lax
from jax.experimental import pallas as pl
from jax.experimental.pallas import tpu as pltpu
from jax.experimental.pallas import tpu_sc as plsc

NC = 2
NS = 16
CH = 128


def _sc_mesh():
    return plsc.VectorSubcoreMesh(core_axis_name="c", subcore_axis_name="s")


@functools.lru_cache(maxsize=None)
def _make_deg_kernel(NP: int, n_chunks: int, D: int):
    rows = NP // NS

    @functools.partial(
        pl.kernel,
        out_type=jax.ShapeDtypeStruct((2, NP, D), jnp.float32),
        mesh=_sc_mesh(),
        scratch_types=[
            pltpu.VMEM((n_chunks, CH), jnp.int32),
            pltpu.VMEM((CH, D), jnp.float32),
            pltpu.VMEM_SHARED((NP, D), jnp.float32),
            pltpu.SemaphoreType.DMA,
        ],
    )
    def deg_kernel(edges_h, zeros_h, ones_h, out_h, idx_v, val_v, acc_sh, sem):
        c = lax.axis_index("c")
        s = lax.axis_index("s")
        pltpu.sync_copy(zeros_h, val_v)
        off = 0
        while off < rows:
            n = min(CH, rows - off)
            pltpu.sync_copy(val_v.at[pl.ds(0, n)],
                            acc_sh.at[pl.ds(s * rows + off, n)])
            off += n
        pltpu.sync_copy(edges_h.at[c, s], idx_v)
        pltpu.sync_copy(ones_h, val_v)
        plsc.subcore_barrier()

        K = 4

        for j in range(K):
            pltpu.async_copy(val_v, acc_sh.at[idx_v.at[j]], sem, add=True)

        def body(j, carry):
            pltpu.make_async_copy(
                val_v, acc_sh.at[idx_v.at[0]], sem).wait()
            pltpu.async_copy(val_v, acc_sh.at[idx_v.at[j + K]], sem, add=True)
            return carry

        lax.fori_loop(0, n_chunks - K, body, 0)
        for j in range(K):
            pltpu.make_async_copy(
                val_v, acc_sh.at[idx_v.at[0]], sem).wait()
        plsc.subcore_barrier()
        pltpu.sync_copy(acc_sh.at[pl.ds(s * rows, rows)],
                        out_h.at[c, pl.ds(s * rows, rows)])

    return deg_kernel


@functools.lru_cache(maxsize=None)
def _make_agg_kernel(NP: int, n_chunks: int, D: int):
    rows = NP // NS
    IB = min(n_chunks, 40)
    blocks = []
    st = 0
    while st < n_chunks:
        blocks.append((st, min(IB, n_chunks - st)))
        st += IB

    @functools.partial(
        pl.kernel,
        out_type=jax.ShapeDtypeStruct((2, NP, D), jnp.float32),
        mesh=_sc_mesh(),
        scratch_types=[
            pltpu.VMEM((IB, CH), jnp.int32),
            pltpu.VMEM((IB, CH), jnp.int32),
            pltpu.VMEM((CH, D), jnp.float32),
            pltpu.VMEM((CH, D), jnp.float32),
            pltpu.VMEM_SHARED((NP, D), jnp.float32),
            pltpu.SemaphoreType.DMA,
            pltpu.SemaphoreType.DMA,
            pltpu.SemaphoreType.DMA,
            pltpu.SemaphoreType.DMA,
        ],
    )
    def agg_kernel(table_h, src_h, dst_h, zeros_h, out_h,
                   si_v, di_v, b0_v, b1_v, acc_sh, gs0, gs1, ss0, ss1):
        c = lax.axis_index("c")
        s = lax.axis_index("s")
        wid = c * NS + s
        pltpu.sync_copy(zeros_h, b0_v)
        off = 0
        while off < rows:
            n = min(CH, rows - off)
            pltpu.sync_copy(b0_v.at[pl.ds(0, n)],
                            acc_sh.at[pl.ds(s * rows + off, n)])
            off += n
        plsc.subcore_barrier()

        bufs = (b0_v, b1_v)
        gsems = (gs0, gs1)
        ssems = (ss0, ss1)

        def wait_g(b):
            pltpu.make_async_copy(
                table_h.at[si_v.at[b]], bufs[b], gsems[b]).wait()

        def wait_s(b):
            pltpu.make_async_copy(
                bufs[b], acc_sh.at[di_v.at[b]], ssems[b]).wait()

        for st_, nb in blocks:
            pltpu.sync_copy(src_h.at[wid, pl.ds(st_, nb)], si_v.at[pl.ds(0, nb)])
            pltpu.sync_copy(dst_h.at[wid, pl.ds(st_, nb)], di_v.at[pl.ds(0, nb)])
            for b in range(2):
                pltpu.async_copy(table_h.at[si_v.at[b]], bufs[b], gsems[b])

            def body(j, carry):
                ch0 = j * 2
                for b in range(2):
                    wait_g(b)
                    pltpu.async_copy(
                        bufs[b], acc_sh.at[di_v.at[ch0 + b]], ssems[b],
                        add=True)
                for b in range(2):
                    wait_s(b)
                    pltpu.async_copy(
                        table_h.at[si_v.at[ch0 + b + 2]], bufs[b], gsems[b])
                return carry

            lax.fori_loop(0, nb // 2 - 1, body, 0)
            for b in range(2):
                wait_g(b)
                pltpu.async_copy(
                    bufs[b], acc_sh.at[di_v.at[nb - 2 + b]], ssems[b],
                    add=True)
            for b in range(2):
                wait_s(b)
        plsc.subcore_barrier()
        pltpu.sync_copy(acc_sh.at[pl.ds(s * rows, rows)],
                        out_h.at[c, pl.ds(s * rows, rows)])

    return agg_kernel


def _norm_col(deg_blk):
    d = deg_blk[:, 0:1]
    return jnp.where(d > 0, lax.rsqrt(jnp.maximum(d, 1e-12)), 0.0)


def _leaky(h):
    return jnp.where(h >= 0, h, 0.01 * h)


def _xs_body(x_ref, deg_ref, o_ref):
    o_ref[...] = x_ref[...] * _norm_col(deg_ref[...])


def _m1_body(agg_ref, degs_ref, w_ref, b_ref, o_ref):
    a = agg_ref[0] + agg_ref[1]
    degs = degs_ref[...]
    nd = jnp.where(degs[1][:, 0:1] > 0,
                   lax.rsqrt(jnp.maximum(degs[1][:, 0:1], 1e-12)), 0.0)
    ns = jnp.where(degs[0][:, 0:1] > 0,
                   lax.rsqrt(jnp.maximum(degs[0][:, 0:1], 1e-12)), 0.0)
    h = jnp.dot(a * nd, w_ref[...], preferred_element_type=jnp.float32)
    o_ref[...] = _leaky(h + b_ref[...]) * ns


def _m2_body(agg_ref, degs_ref, w_ref, b_ref, wl_ref, bl_ref, o_ref):
    a = agg_ref[0] + agg_ref[1]
    degs = degs_ref[...]
    nd = jnp.where(degs[1][:, 0:1] > 0,
                   lax.rsqrt(jnp.maximum(degs[1][:, 0:1], 1e-12)), 0.0)
    h = _leaky(jnp.dot(a * nd, w_ref[...],
                       preferred_element_type=jnp.float32) + b_ref[...])
    o_ref[...] = jnp.dot(h, wl_ref[...],
                         preferred_element_type=jnp.float32) + bl_ref[...]


@functools.lru_cache(maxsize=None)
def _make_tc_kernels(NP: int, D: int):
    R = NP // 16
    grid = (NP // R,)
    row_spec = pl.BlockSpec((R, D), lambda i: (i, 0))
    deg1_spec = pl.BlockSpec((R, D), lambda i: (i, 0))
    degs_spec = pl.BlockSpec((2, R, D), lambda i: (0, i, 0))
    agg_spec = pl.BlockSpec((2, R, D), lambda i: (0, i, 0))
    w_spec = pl.BlockSpec((D, D), lambda i: (0, 0))
    b_spec = pl.BlockSpec((1, D), lambda i: (0, 0))
    out = jax.ShapeDtypeStruct((NP, D), jnp.float32)

    xs_call = pl.pallas_call(
        _xs_body, grid=grid, in_specs=[row_spec, deg1_spec],
        out_specs=row_spec, out_shape=out)
    m1_call = pl.pallas_call(
        _m1_body, grid=grid,
        in_specs=[agg_spec, degs_spec, w_spec, b_spec],
        out_specs=row_spec, out_shape=out)
    m2_call = pl.pallas_call(
        _m2_body, grid=grid,
        in_specs=[agg_spec, degs_spec, w_spec, b_spec, w_spec, b_spec],
        out_specs=row_spec, out_shape=out)
    return xs_call, m1_call, m2_call


def kernel(x, edge_index, W1, b1, W2, b2, Wl, bl):
    N, D = x.shape
    E = edge_index.shape[1]
    C = Wl.shape[1]

    NP = ((N + 1 + 127) // 128) * 128
    EPAD = ((E + 2 * NC * NS * CH - 1) // (2 * NC * NS * CH)) * (2 * NC * NS * CH)
    n_agg = EPAD // (NC * NS * CH)
    n_deg = EPAD // (NS * CH)

    npad = EPAD - E
    tr = N + (jnp.arange(npad, dtype=jnp.int32) % (NP - N))
    e_pad = jnp.concatenate([edge_index, jnp.stack([tr, tr])], axis=1)
    e_deg = e_pad.reshape(2, NS, n_deg, CH)
    src_agg = e_pad[0].reshape(NC * NS, n_agg, CH)
    dst_agg = e_pad[1].reshape(NC * NS, n_agg, CH)

    xp = jnp.zeros((NP, D), jnp.float32).at[:N].set(x)
    ones_c = jnp.ones((CH, D), jnp.float32)
    zeros_c = jnp.zeros((CH, D), jnp.float32)
    b1p = b1.reshape(1, D)
    b2p = b2.reshape(1, D)
    Wlp = jnp.zeros((D, D), jnp.float32).at[:, :C].set(Wl)
    blp = jnp.zeros((1, D), jnp.float32).at[0, :C].set(bl)

    deg_k = _make_deg_kernel(NP, n_deg, D)
    agg_k = _make_agg_kernel(NP, n_agg, D)
    xs_call, m1_call, m2_call = _make_tc_kernels(NP, D)

    degs = deg_k(e_deg, zeros_c, ones_c)
    xs = xs_call(xp, degs[0])
    agg1 = agg_k(xs, src_agg, dst_agg, zeros_c)
    h1s = m1_call(agg1, degs, W1, b1p)
    agg2 = agg_k(h1s, src_agg, dst_agg, zeros_c)
    outp = m2_call(agg2, degs, W2, b2p, Wlp, blp)
    return outp[:N, :C]


# speedup vs baseline: 2.4728x; 1.0014x over previous
import functools

import jax
import jax.numpy as jnp
from jax import lax
from jax.experimental import pallas as pl
from jax.experimental.pallas import tpu as pltpu
from jax.experimental.pallas import tpu_sc as plsc

NC = 2
NS = 16
CH = 128


def _sc_mesh():
    return plsc.VectorSubcoreMesh(core_axis_name="c", subcore_axis_name="s")


@functools.lru_cache(maxsize=None)
def _make_deg_kernel(NP: int, n_chunks: int, D: int):
    rows = NP // NS

    @functools.partial(
        pl.kernel,
        out_type=jax.ShapeDtypeStruct((2, NP, D), jnp.float32),
        mesh=_sc_mesh(),
        scratch_types=[
            pltpu.VMEM((n_chunks, CH), jnp.int32),
            pltpu.VMEM((CH, D), jnp.float32),
            pltpu.VMEM_SHARED((NP, D), jnp.float32),
            pltpu.SemaphoreType.DMA,
        ],
    )
    def deg_kernel(edges_h, zeros_h, ones_h, out_h, idx_v, val_v, acc_sh, sem):
        c = lax.axis_index("c")
        s = lax.axis_index("s")
        pltpu.sync_copy(zeros_h, val_v)
        off = 0
        while off < rows:
            n = min(CH, rows - off)
            pltpu.sync_copy(val_v.at[pl.ds(0, n)],
                            acc_sh.at[pl.ds(s * rows + off, n)])
            off += n
        pltpu.sync_copy(edges_h.at[c, s], idx_v)
        pltpu.sync_copy(ones_h, val_v)
        plsc.subcore_barrier()

        K = 4

        for j in range(K):
            pltpu.async_copy(val_v, acc_sh.at[idx_v.at[j]], sem, add=True)

        def body(j, carry):
            pltpu.make_async_copy(
                val_v, acc_sh.at[idx_v.at[0]], sem).wait()
            pltpu.async_copy(val_v, acc_sh.at[idx_v.at[j + K]], sem, add=True)
            return carry

        lax.fori_loop(0, n_chunks - K, body, 0)
        for j in range(K):
            pltpu.make_async_copy(
                val_v, acc_sh.at[idx_v.at[0]], sem).wait()
        plsc.subcore_barrier()
        pltpu.sync_copy(acc_sh.at[pl.ds(s * rows, rows)],
                        out_h.at[c, pl.ds(s * rows, rows)])

    return deg_kernel


@functools.lru_cache(maxsize=None)
def _make_agg_kernel(NP: int, n_chunks: int, D: int):
    rows = NP // NS
    IB = min(n_chunks, 40)
    blocks = []
    st = 0
    while st < n_chunks:
        blocks.append((st, min(IB, n_chunks - st)))
        st += IB

    @functools.partial(
        pl.kernel,
        out_type=jax.ShapeDtypeStruct((2, NP, D), jnp.float32),
        mesh=_sc_mesh(),
        scratch_types=[
            pltpu.VMEM((IB, CH), jnp.int32),
            pltpu.VMEM((IB, CH), jnp.int32),
            pltpu.VMEM((CH, D), jnp.float32),
            pltpu.VMEM((CH, D), jnp.float32),
            pltpu.VMEM_SHARED((NP, D), jnp.float32),
            pltpu.SemaphoreType.DMA,
            pltpu.SemaphoreType.DMA,
            pltpu.SemaphoreType.DMA,
            pltpu.SemaphoreType.DMA,
        ],
    )
    def agg_kernel(table_h, src_h, dst_h, zeros_h, out_h,
                   si_v, di_v, b0_v, b1_v, acc_sh, gs0, gs1, ss0, ss1):
        c = lax.axis_index("c")
        s = lax.axis_index("s")
        wid = c * NS + s
        pltpu.sync_copy(zeros_h, b0_v)
        off = 0
        while off < rows:
            n = min(CH, rows - off)
            pltpu.sync_copy(b0_v.at[pl.ds(0, n)],
                            acc_sh.at[pl.ds(s * rows + off, n)])
            off += n
        plsc.subcore_barrier()

        bufs = (b0_v, b1_v)
        gsems = (gs0, gs1)
        ssems = (ss0, ss1)

        def wait_g(b):
            pltpu.make_async_copy(
                table_h.at[si_v.at[b]], bufs[b], gsems[b]).wait()

        def wait_s(b):
            pltpu.make_async_copy(
                bufs[b], acc_sh.at[di_v.at[b]], ssems[b]).wait()

        for st_, nb in blocks:
            pltpu.sync_copy(src_h.at[wid, pl.ds(st_, nb)], si_v.at[pl.ds(0, nb)])
            pltpu.sync_copy(dst_h.at[wid, pl.ds(st_, nb)], di_v.at[pl.ds(0, nb)])
            for b in range(2):
                pltpu.async_copy(table_h.at[si_v.at[b]], bufs[b], gsems[b])

            def body(j, carry):
                ch0 = j * 2
                for b in range(2):
                    wait_g(b)
                    pltpu.async_copy(
                        bufs[b], acc_sh.at[di_v.at[ch0 + b]], ssems[b],
                        add=True)
                for b in range(2):
                    wait_s(b)
                    pltpu.async_copy(
                        table_h.at[si_v.at[ch0 + b + 2]], bufs[b], gsems[b])
                return carry

            lax.fori_loop(0, nb // 2 - 1, body, 0)
            for b in range(2):
                wait_g(b)
                pltpu.async_copy(
                    bufs[b], acc_sh.at[di_v.at[nb - 2 + b]], ssems[b],
                    add=True)
            for b in range(2):
                wait_s(b)
        plsc.subcore_barrier()
        pltpu.sync_copy(acc_sh.at[pl.ds(s * rows, rows)],
                        out_h.at[c, pl.ds(s * rows, rows)])

    return agg_kernel


def _leaky(h):
    return jnp.where(h >= 0, h, 0.01 * h)


def _norm1(d):
    return jnp.where(d > 0, lax.rsqrt(jnp.maximum(d, 1e-12)), 0.0)


def _xs_body(x_ref, degs_ref, o_ref, n_ref):
    degs = degs_ref[...]
    ns = _norm1(degs[0][:, 0:1])
    nd = _norm1(degs[1][:, 0:1])
    o_ref[...] = x_ref[...] * ns
    n_ref[0] = jnp.broadcast_to(ns, n_ref.shape[1:])
    n_ref[1] = jnp.broadcast_to(nd, n_ref.shape[1:])


def _m1_body(agg_ref, n_ref, w_ref, b_ref, o_ref):
    a = agg_ref[0] + agg_ref[1]
    ns = n_ref[0][:, 0:1]
    nd = n_ref[1][:, 0:1]
    h = jnp.dot(a * nd, w_ref[...], preferred_element_type=jnp.float32)
    o_ref[...] = _leaky(h + b_ref[...]) * ns


def _m2_body(agg_ref, n_ref, w_ref, b_ref, wl_ref, bl_ref, o_ref):
    a = agg_ref[0] + agg_ref[1]
    nd = n_ref[1][:, 0:1]
    h = _leaky(jnp.dot(a * nd, w_ref[...],
                       preferred_element_type=jnp.float32) + b_ref[...])
    o_ref[...] = jnp.dot(h, wl_ref[...],
                         preferred_element_type=jnp.float32) + bl_ref[...]


@functools.lru_cache(maxsize=None)
def _make_tc_kernels(NP: int, D: int):
    R = NP // 16
    grid = (NP // R,)
    row_spec = pl.BlockSpec((R, D), lambda i: (i, 0))
    degs_spec = pl.BlockSpec((2, R, D), lambda i: (0, i, 0))
    agg_spec = pl.BlockSpec((2, R, D), lambda i: (0, i, 0))
    nrm_spec = pl.BlockSpec((2, R, 8), lambda i: (0, i, 0))
    w_spec = pl.BlockSpec((D, D), lambda i: (0, 0))
    b_spec = pl.BlockSpec((1, D), lambda i: (0, 0))
    out = jax.ShapeDtypeStruct((NP, D), jnp.float32)
    nrm_out = jax.ShapeDtypeStruct((2, NP, 8), jnp.float32)

    xs_call = pl.pallas_call(
        _xs_body, grid=grid, in_specs=[row_spec, degs_spec],
        out_specs=[row_spec, nrm_spec], out_shape=[out, nrm_out])
    m1_call = pl.pallas_call(
        _m1_body, grid=grid,
        in_specs=[agg_spec, nrm_spec, w_spec, b_spec],
        out_specs=row_spec, out_shape=out)
    m2_call = pl.pallas_call(
        _m2_body, grid=grid,
        in_specs=[agg_spec, nrm_spec, w_spec, b_spec, w_spec, b_spec],
        out_specs=row_spec, out_shape=out)
    return xs_call, m1_call, m2_call


def kernel(x, edge_index, W1, b1, W2, b2, Wl, bl):
    N, D = x.shape
    E = edge_index.shape[1]
    C = Wl.shape[1]

    NP = ((N + 1 + 127) // 128) * 128
    EPAD = ((E + 2 * NC * NS * CH - 1) // (2 * NC * NS * CH)) * (2 * NC * NS * CH)
    n_agg = EPAD // (NC * NS * CH)
    n_deg = EPAD // (NS * CH)

    npad = EPAD - E
    tr = N + (jnp.arange(npad, dtype=jnp.int32) % (NP - N))
    e_pad = jnp.concatenate([edge_index, jnp.stack([tr, tr])], axis=1)
    e_deg = e_pad.reshape(2, NS, n_deg, CH)
    src_agg = e_pad[0].reshape(NC * NS, n_agg, CH)
    dst_agg = e_pad[1].reshape(NC * NS, n_agg, CH)

    xp = jnp.zeros((NP, D), jnp.float32).at[:N].set(x)
    ones_c = jnp.ones((CH, D), jnp.float32)
    zeros_c = jnp.zeros((CH, D), jnp.float32)
    b1p = b1.reshape(1, D)
    b2p = b2.reshape(1, D)
    Wlp = jnp.zeros((D, D), jnp.float32).at[:, :C].set(Wl)
    blp = jnp.zeros((1, D), jnp.float32).at[0, :C].set(bl)

    deg_k = _make_deg_kernel(NP, n_deg, D)
    agg_k = _make_agg_kernel(NP, n_agg, D)
    xs_call, m1_call, m2_call = _make_tc_kernels(NP, D)

    degs = deg_k(e_deg, zeros_c, ones_c)
    xs, nrms = xs_call(xp, degs)
    agg1 = agg_k(xs, src_agg, dst_agg, zeros_c)
    h1s = m1_call(agg1, nrms, W1, b1p)
    agg2 = agg_k(h1s, src_agg, dst_agg, zeros_c)
    outp = m2_call(agg2, nrms, W2, b2p, Wlp, blp)
    return outp[:N, :C]
